# baseline (speedup 1.0000x reference)
import functools

import jax
import jax.numpy as jnp
from jax import lax
from jax.experimental import pallas as pl
from jax.experimental.pallas import tpu as pltpu
from jax.experimental.pallas import tpu_sc as plsc

N = 10000
E = 160000
D = 256
H = 128
G = 64
OUT = 128

NC = 2
NS = 16
CHUNK = 50
NCHUNK = E // (CHUNK * NS)
NBUF = 5

WCHUNK = 8
RB = 640
NP = NS * RB
WB = RB // WCHUNK

SB = 40
NSUPER = NCHUNK // SB

NB = 2000
GRID = N // NB

_mesh = plsc.VectorSubcoreMesh(
    core_axis_name="c", subcore_axis_name="s", num_cores=NC, num_subcores=NS)


@functools.partial(
    pl.kernel,
    out_type=jax.ShapeDtypeStruct((NC, NP, H), jnp.float32),
    mesh=_mesh,
    scratch_types=[
        pltpu.VMEM((2, SB, CHUNK), jnp.int32),
        pltpu.VMEM((NBUF, CHUNK, H), jnp.float32),
        pltpu.VMEM((8, H), jnp.float32),
        pltpu.VMEM_SHARED((NP, H), jnp.float32),
        pltpu.SemaphoreType.DMA,
        pltpu.SemaphoreType.DMA,
        pltpu.SemaphoreType.DMA,
        pltpu.SemaphoreType.DMA,
        pltpu.SemaphoreType.DMA,
    ],
)
def _sc_agg(h_both, idx4d, agg_both, idx_v, rowsbuf, stage, acc,
            sem0, sem1, sem2, sem3, sem4):
    s = lax.axis_index("s")
    c = lax.axis_index("c")
    h_ref = h_both.at[c]
    out_ref = agg_both.at[c]
    base = s * RB
    rows = tuple(rowsbuf.at[b] for b in range(NBUF))
    sems = (sem0, sem1, sem2, sem3, sem4)

    zero16 = jnp.zeros((16,), jnp.float32)

    @pl.loop(0, WCHUNK)
    def _(w):
        for l in range(H // 16):
            stage[w, pl.ds(l * 16, 16)] = zero16

    @pl.loop(0, WB)
    def _(w):
        pltpu.sync_copy(stage, acc.at[pl.ds(base + w * WCHUNK, WCHUNK)])

    plsc.subcore_barrier()

    def _drain(b):
        pltpu.make_async_copy(h_ref.at[idx_v.at[0, 0]], rows[b], sems[b]).wait()

    @pl.loop(0, NSUPER)
    def _(m):
        pltpu.sync_copy(idx4d.at[:, s, pl.ds(m * SB, SB)], idx_v)
        for b in range(NBUF):
            pltpu.async_copy(h_ref.at[idx_v.at[0, b]], rows[b], sems[b])

        @pl.loop(0, SB - NBUF, step=NBUF)
        def _(j):
            for b in range(NBUF):
                _drain(b)
                pltpu.sync_copy(rows[b], acc.at[idx_v.at[1, j + b]], add=True)
                pltpu.async_copy(h_ref.at[idx_v.at[0, j + b + NBUF]], rows[b],
                                 sems[b])

        for b in range(NBUF):
            _drain(b)
            pltpu.sync_copy(rows[b], acc.at[idx_v.at[1, SB - NBUF + b]],
                            add=True)

    plsc.subcore_barrier()

    @pl.loop(0, WB)
    def _(w):
        off = base + w * WCHUNK
        pltpu.sync_copy(acc.at[pl.ds(off, WCHUNK)], out_ref.at[pl.ds(off, WCHUNK)])


def _tc_layer_body(hlo_ref, hhi_ref, alo_ref, ahi_ref, W_ref, b_ref, o_ref):
    h = jnp.concatenate(
        [hlo_ref[0] + alo_ref[0], hhi_ref[0] + ahi_ref[0]], axis=1)
    z = jnp.dot(h, W_ref[...], preferred_element_type=jnp.float32,
                precision=lax.Precision.HIGHEST) + b_ref[...]
    r = jnp.maximum(z, 0.0)
    o_ref[0] = r[:, :H]
    o_ref[1] = r[:, H:]


def _tc_layer(h_both, a_both, W, b):
    return pl.pallas_call(
        _tc_layer_body,
        grid=(GRID,),
        in_specs=[
            pl.BlockSpec((1, NB, H), lambda i: (0, i, 0)),
            pl.BlockSpec((1, NB, H), lambda i: (1, i, 0)),
            pl.BlockSpec((1, NB, H), lambda i: (0, i, 0)),
            pl.BlockSpec((1, NB, H), lambda i: (1, i, 0)),
            pl.BlockSpec((D, D), lambda i: (0, 0)),
            pl.BlockSpec((1, D), lambda i: (0, 0)),
        ],
        out_specs=pl.BlockSpec((NC, NB, H), lambda i: (0, i, 0)),
        out_shape=jax.ShapeDtypeStruct((NC, N, H), jnp.float32),
    )(h_both, h_both, a_both, a_both, W, b.reshape(1, D))


def _tc_final_body(hlo_ref, hhi_ref, alo_ref, ahi_ref, W2_ref, b2_ref,
                   W3_ref, b3_ref, batch_ref, out_ref, pooled, cnt):
    i = pl.program_id(0)

    @pl.when(i == 0)
    def _():
        pooled[...] = jnp.zeros_like(pooled)
        cnt[...] = jnp.zeros_like(cnt)

    h = jnp.concatenate(
        [hlo_ref[0] + alo_ref[0], hhi_ref[0] + ahi_ref[0]], axis=1)
    z = jnp.dot(h, W2_ref[...], preferred_element_type=jnp.float32,
                precision=lax.Precision.HIGHEST) + b2_ref[...]
    r = jnp.maximum(z, 0.0)

    row = batch_ref[0]
    ids = lax.broadcasted_iota(jnp.int32, (G, NB), 0)
    onehotT = (ids == row).astype(jnp.float32)
    dn = (((1,), (0,)), ((), ()))
    pooled[...] += lax.dot_general(onehotT, r, dn,
                                   preferred_element_type=jnp.float32,
                                   precision=lax.Precision.HIGHEST)
    cnt[...] += lax.dot_general(onehotT, jnp.ones_like(r), dn,
                                preferred_element_type=jnp.float32,
                                precision=lax.Precision.HIGHEST)

    @pl.when(i == pl.num_programs(0) - 1)
    def _():
        pm = pooled[...] / jnp.maximum(cnt[...], 1.0)
        out_ref[...] = jnp.dot(pm, W3_ref[...],
                               preferred_element_type=jnp.float32,
                               precision=lax.Precision.HIGHEST) + b3_ref[...]


def _tc_final(h_both, a_both, W2, b2, W3, b3, batch):
    return pl.pallas_call(
        _tc_final_body,
        grid=(GRID,),
        in_specs=[
            pl.BlockSpec((1, NB, H), lambda i: (0, i, 0)),
            pl.BlockSpec((1, NB, H), lambda i: (1, i, 0)),
            pl.BlockSpec((1, NB, H), lambda i: (0, i, 0)),
            pl.BlockSpec((1, NB, H), lambda i: (1, i, 0)),
            pl.BlockSpec((D, D), lambda i: (0, 0)),
            pl.BlockSpec((1, D), lambda i: (0, 0)),
            pl.BlockSpec((D, OUT), lambda i: (0, 0)),
            pl.BlockSpec((1, OUT), lambda i: (0, 0)),
            pl.BlockSpec((1, 1, NB), lambda i: (i, 0, 0)),
        ],
        out_specs=pl.BlockSpec((G, OUT), lambda i: (0, 0)),
        out_shape=jax.ShapeDtypeStruct((G, OUT), jnp.float32),
        scratch_shapes=[pltpu.VMEM((G, D), jnp.float32),
                        pltpu.VMEM((G, D), jnp.float32)],
    )(h_both, h_both, a_both, a_both, W2, b2.reshape(1, D), W3,
      b3.reshape(1, OUT), batch.reshape(GRID, 1, NB))


def kernel(x, edge_index, batch, W1, b1, W2, b2, W3, b3):
    idx2d = edge_index.reshape(2, NS, NCHUNK, CHUNK)
    x_both = jnp.stack([x[:, :H], x[:, H:]])

    a1 = _sc_agg(x_both, idx2d)
    h1 = _tc_layer(x_both, a1, W1, b1)
    a2 = _sc_agg(h1, idx2d)
    return _tc_final(h1, a2, W2, b2, W3, b3, batch)


# speedup vs baseline: 1.3222x; 1.3222x over previous
import functools

import jax
import jax.numpy as jnp
from jax import lax
from jax.experimental import pallas as pl
from jax.experimental.pallas import tpu as pltpu
from jax.experimental.pallas import tpu_sc as plsc

N = 10000
E = 160000
D = 256
H = 128
G = 64
OUT = 128

NC = 2
NS = 16
CHUNK = 50
NCHUNK = E // (CHUNK * NS)
NBUF = 5

ZCH = 16
RB = 640
NP = NS * RB
ZB = RB // ZCH

SB = 40
NSUPER = NCHUNK // SB

NB = 2000
GRID = N // NB

_mesh = plsc.VectorSubcoreMesh(
    core_axis_name="c", subcore_axis_name="s", num_cores=NC, num_subcores=NS)


@functools.partial(
    pl.kernel,
    out_type=jax.ShapeDtypeStruct((NC, NP, H), jnp.float32),
    mesh=_mesh,
    scratch_types=[
        pltpu.VMEM((2, SB, CHUNK), jnp.int32),
        pltpu.VMEM((NBUF, CHUNK, H), jnp.float32),
        pltpu.VMEM((ZCH, H), jnp.float32),
        pltpu.VMEM_SHARED((NP, H), jnp.float32),
        pltpu.SemaphoreType.DMA,
        pltpu.SemaphoreType.DMA,
        pltpu.SemaphoreType.DMA,
        pltpu.SemaphoreType.DMA,
        pltpu.SemaphoreType.DMA,
    ],
)
def _sc_agg(h_both, idx4d, agg_both, idx_v, rowsbuf, stage, acc,
            sem0, sem1, sem2, sem3, sem4):
    s = lax.axis_index("s")
    c = lax.axis_index("c")
    h_ref = h_both.at[c]
    out_ref = agg_both.at[c]
    base = s * RB
    rows = tuple(rowsbuf.at[b] for b in range(NBUF))
    sems = (sem0, sem1, sem2, sem3, sem4)

    zero16 = jnp.zeros((16,), jnp.float32)

    @pl.loop(0, ZCH)
    def _(w):
        for l in range(H // 16):
            stage[w, pl.ds(l * 16, 16)] = zero16

    @pl.loop(0, ZB)
    def _(w):
        pltpu.sync_copy(stage, acc.at[pl.ds(base + w * ZCH, ZCH)])

    plsc.subcore_barrier()

    def _drain(b):
        pltpu.make_async_copy(h_ref.at[idx_v.at[0, 0]], rows[b], sems[b]).wait()

    @pl.loop(0, NSUPER)
    def _(m):
        pltpu.sync_copy(idx4d.at[:, s, pl.ds(m * SB, SB)], idx_v)
        for b in range(NBUF):
            pltpu.async_copy(h_ref.at[idx_v.at[0, b]], rows[b], sems[b])

        @pl.loop(0, SB - NBUF, step=NBUF)
        def _(j):
            for b in range(NBUF):
                _drain(b)
                pltpu.sync_copy(rows[b], acc.at[idx_v.at[1, j + b]], add=True)
                pltpu.async_copy(h_ref.at[idx_v.at[0, j + b + NBUF]], rows[b],
                                 sems[b])

        for b in range(NBUF):
            _drain(b)
            pltpu.sync_copy(rows[b], acc.at[idx_v.at[1, SB - NBUF + b]],
                            add=True)

    plsc.subcore_barrier()

    pltpu.sync_copy(acc.at[pl.ds(base, RB)], out_ref.at[pl.ds(base, RB)])


def _tc_layer_body(hlo_ref, hhi_ref, alo_ref, ahi_ref, W_ref, b_ref, o_ref):
    h = jnp.concatenate(
        [hlo_ref[0] + alo_ref[0], hhi_ref[0] + ahi_ref[0]], axis=1)
    z = jnp.dot(h, W_ref[...], preferred_element_type=jnp.float32,
                precision=lax.Precision.HIGHEST) + b_ref[...]
    r = jnp.maximum(z, 0.0)
    o_ref[0] = r[:, :H]
    o_ref[1] = r[:, H:]


def _tc_layer(h_both, a_both, W, b):
    return pl.pallas_call(
        _tc_layer_body,
        grid=(GRID,),
        in_specs=[
            pl.BlockSpec((1, NB, H), lambda i: (0, i, 0)),
            pl.BlockSpec((1, NB, H), lambda i: (1, i, 0)),
            pl.BlockSpec((1, NB, H), lambda i: (0, i, 0)),
            pl.BlockSpec((1, NB, H), lambda i: (1, i, 0)),
            pl.BlockSpec((D, D), lambda i: (0, 0)),
            pl.BlockSpec((1, D), lambda i: (0, 0)),
        ],
        out_specs=pl.BlockSpec((NC, NB, H), lambda i: (0, i, 0)),
        out_shape=jax.ShapeDtypeStruct((NC, N, H), jnp.float32),
    )(h_both, h_both, a_both, a_both, W, b.reshape(1, D))


def _tc_final_body(hlo_ref, hhi_ref, alo_ref, ahi_ref, W2_ref, b2_ref,
                   W3_ref, b3_ref, batch_ref, out_ref, pooled, cnt):
    i = pl.program_id(0)

    @pl.when(i == 0)
    def _():
        pooled[...] = jnp.zeros_like(pooled)
        cnt[...] = jnp.zeros_like(cnt)

    h = jnp.concatenate(
        [hlo_ref[0] + alo_ref[0], hhi_ref[0] + ahi_ref[0]], axis=1)
    z = jnp.dot(h, W2_ref[...], preferred_element_type=jnp.float32,
                precision=lax.Precision.HIGHEST) + b2_ref[...]
    r = jnp.maximum(z, 0.0)

    row = batch_ref[0]
    ids = lax.broadcasted_iota(jnp.int32, (G, NB), 0)
    onehotT = (ids == row).astype(jnp.float32)
    dn = (((1,), (0,)), ((), ()))
    pooled[...] += lax.dot_general(onehotT, r, dn,
                                   preferred_element_type=jnp.float32,
                                   precision=lax.Precision.HIGHEST)
    cnt[...] += lax.dot_general(onehotT, jnp.ones_like(r), dn,
                                preferred_element_type=jnp.float32,
                                precision=lax.Precision.HIGHEST)

    @pl.when(i == pl.num_programs(0) - 1)
    def _():
        pm = pooled[...] / jnp.maximum(cnt[...], 1.0)
        out_ref[...] = jnp.dot(pm, W3_ref[...],
                               preferred_element_type=jnp.float32,
                               precision=lax.Precision.HIGHEST) + b3_ref[...]


def _tc_final(h_both, a_both, W2, b2, W3, b3, batch):
    return pl.pallas_call(
        _tc_final_body,
        grid=(GRID,),
        in_specs=[
            pl.BlockSpec((1, NB, H), lambda i: (0, i, 0)),
            pl.BlockSpec((1, NB, H), lambda i: (1, i, 0)),
            pl.BlockSpec((1, NB, H), lambda i: (0, i, 0)),
            pl.BlockSpec((1, NB, H), lambda i: (1, i, 0)),
            pl.BlockSpec((D, D), lambda i: (0, 0)),
            pl.BlockSpec((1, D), lambda i: (0, 0)),
            pl.BlockSpec((D, OUT), lambda i: (0, 0)),
            pl.BlockSpec((1, OUT), lambda i: (0, 0)),
            pl.BlockSpec((1, 1, NB), lambda i: (i, 0, 0)),
        ],
        out_specs=pl.BlockSpec((G, OUT), lambda i: (0, 0)),
        out_shape=jax.ShapeDtypeStruct((G, OUT), jnp.float32),
        scratch_shapes=[pltpu.VMEM((G, D), jnp.float32),
                        pltpu.VMEM((G, D), jnp.float32)],
    )(h_both, h_both, a_both, a_both, W2, b2.reshape(1, D), W3,
      b3.reshape(1, OUT), batch.reshape(GRID, 1, NB))


def kernel(x, edge_index, batch, W1, b1, W2, b2, W3, b3):
    idx2d = edge_index.reshape(2, NS, NCHUNK, CHUNK)
    x_both = jnp.stack([x[:, :H], x[:, H:]])

    a1 = _sc_agg(x_both, idx2d)
    h1 = _tc_layer(x_both, a1, W1, b1)
    a2 = _sc_agg(h1, idx2d)
    return _tc_final(h1, a2, W2, b2, W3, b3, batch)


# speedup vs baseline: 1.3557x; 1.0253x over previous
import functools

import jax
import jax.numpy as jnp
from jax import lax
from jax.experimental import pallas as pl
from jax.experimental.pallas import tpu as pltpu
from jax.experimental.pallas import tpu_sc as plsc

N = 10000
E = 160000
D = 256
H = 128
G = 64
OUT = 128

NC = 2
NS = 16
CHUNK = 50
NCHUNK = E // (CHUNK * NS)
NBUF = 5

ZCH = 16
RB = 640
NP = NS * RB
ZB = RB // ZCH

SB = 40
NSUPER = NCHUNK // SB

NB = 2000
GRID = N // NB

_mesh = plsc.VectorSubcoreMesh(
    core_axis_name="c", subcore_axis_name="s", num_cores=NC, num_subcores=NS)


@functools.partial(
    pl.kernel,
    out_type=jax.ShapeDtypeStruct((NC, NP, H), jnp.float32),
    mesh=_mesh,
    scratch_types=[
        pltpu.VMEM((2, SB, CHUNK), jnp.int32),
        pltpu.VMEM((NBUF, CHUNK, H), jnp.float32),
        pltpu.VMEM((ZCH, H), jnp.float32),
        pltpu.VMEM_SHARED((NP, H), jnp.float32),
        pltpu.SemaphoreType.DMA,
        pltpu.SemaphoreType.DMA,
        pltpu.SemaphoreType.DMA,
        pltpu.SemaphoreType.DMA,
        pltpu.SemaphoreType.DMA,
        pltpu.SemaphoreType.DMA,
    ],
)
def _sc_agg(h_both, idx4d, agg_both, idx_v, rowsbuf, stage, acc,
            sem0, sem1, sem2, sem3, sem4, zsem):
    s = lax.axis_index("s")
    c = lax.axis_index("c")
    h_ref = h_both.at[c]
    out_ref = agg_both.at[c]
    base = s * RB
    rows = tuple(rowsbuf.at[b] for b in range(NBUF))
    sems = (sem0, sem1, sem2, sem3, sem4)

    pltpu.sync_copy(idx4d.at[:, s, pl.ds(0, SB)], idx_v)
    for b in range(NBUF):
        pltpu.async_copy(h_ref.at[idx_v.at[0, b]], rows[b], sems[b])

    zero16 = jnp.zeros((16,), jnp.float32)

    @pl.loop(0, ZCH)
    def _(w):
        for l in range(H // 16):
            stage[w, pl.ds(l * 16, 16)] = zero16

    @pl.loop(0, ZB)
    def _(w):
        pltpu.async_copy(stage, acc.at[pl.ds(base + w * ZCH, ZCH)], zsem)

    @pl.loop(0, ZB)
    def _(w):
        pltpu.make_async_copy(stage, acc.at[pl.ds(base, ZCH)], zsem).wait()

    plsc.subcore_barrier()

    def _drain(b):
        pltpu.make_async_copy(h_ref.at[idx_v.at[0, 0]], rows[b], sems[b]).wait()

    @pl.loop(0, NSUPER)
    def _(m):
        @pl.when(m > 0)
        def _():
            pltpu.sync_copy(idx4d.at[:, s, pl.ds(m * SB, SB)], idx_v)
            for b in range(NBUF):
                pltpu.async_copy(h_ref.at[idx_v.at[0, b]], rows[b], sems[b])

        @pl.loop(0, SB - NBUF, step=NBUF)
        def _(j):
            for b in range(NBUF):
                _drain(b)
                pltpu.sync_copy(rows[b], acc.at[idx_v.at[1, j + b]], add=True)
                pltpu.async_copy(h_ref.at[idx_v.at[0, j + b + NBUF]], rows[b],
                                 sems[b])

        for b in range(NBUF):
            _drain(b)
            pltpu.sync_copy(rows[b], acc.at[idx_v.at[1, SB - NBUF + b]],
                            add=True)

    plsc.subcore_barrier()

    pltpu.sync_copy(acc.at[pl.ds(base, RB)], out_ref.at[pl.ds(base, RB)])


def _tc_layer_body(hlo_ref, hhi_ref, alo_ref, ahi_ref, W_ref, b_ref, o_ref):
    h = jnp.concatenate(
        [hlo_ref[0] + alo_ref[0], hhi_ref[0] + ahi_ref[0]], axis=1)
    z = jnp.dot(h, W_ref[...], preferred_element_type=jnp.float32,
                precision=lax.Precision.HIGHEST) + b_ref[...]
    r = jnp.maximum(z, 0.0)
    o_ref[0] = r[:, :H]
    o_ref[1] = r[:, H:]


def _tc_layer(h_both, a_both, W, b):
    return pl.pallas_call(
        _tc_layer_body,
        grid=(GRID,),
        in_specs=[
            pl.BlockSpec((1, NB, H), lambda i: (0, i, 0)),
            pl.BlockSpec((1, NB, H), lambda i: (1, i, 0)),
            pl.BlockSpec((1, NB, H), lambda i: (0, i, 0)),
            pl.BlockSpec((1, NB, H), lambda i: (1, i, 0)),
            pl.BlockSpec((D, D), lambda i: (0, 0)),
            pl.BlockSpec((1, D), lambda i: (0, 0)),
        ],
        out_specs=pl.BlockSpec((NC, NB, H), lambda i: (0, i, 0)),
        out_shape=jax.ShapeDtypeStruct((NC, N, H), jnp.float32),
    )(h_both, h_both, a_both, a_both, W, b.reshape(1, D))


def _tc_final_body(hlo_ref, hhi_ref, alo_ref, ahi_ref, W2_ref, b2_ref,
                   W3_ref, b3_ref, batch_ref, out_ref, pooled, cnt):
    i = pl.program_id(0)

    @pl.when(i == 0)
    def _():
        pooled[...] = jnp.zeros_like(pooled)
        cnt[...] = jnp.zeros_like(cnt)

    h = jnp.concatenate(
        [hlo_ref[0] + alo_ref[0], hhi_ref[0] + ahi_ref[0]], axis=1)
    z = jnp.dot(h, W2_ref[...], preferred_element_type=jnp.float32,
                precision=lax.Precision.HIGHEST) + b2_ref[...]
    r = jnp.maximum(z, 0.0)

    row = batch_ref[0]
    ids = lax.broadcasted_iota(jnp.int32, (G, NB), 0)
    onehotT = (ids == row).astype(jnp.float32)
    dn = (((1,), (0,)), ((), ()))
    pooled[...] += lax.dot_general(onehotT, r, dn,
                                   preferred_element_type=jnp.float32,
                                   precision=lax.Precision.HIGHEST)
    cnt[...] += lax.dot_general(onehotT, jnp.ones_like(r), dn,
                                preferred_element_type=jnp.float32,
                                precision=lax.Precision.HIGHEST)

    @pl.when(i == pl.num_programs(0) - 1)
    def _():
        pm = pooled[...] / jnp.maximum(cnt[...], 1.0)
        out_ref[...] = jnp.dot(pm, W3_ref[...],
                               preferred_element_type=jnp.float32,
                               precision=lax.Precision.HIGHEST) + b3_ref[...]


def _tc_final(h_both, a_both, W2, b2, W3, b3, batch):
    return pl.pallas_call(
        _tc_final_body,
        grid=(GRID,),
        in_specs=[
            pl.BlockSpec((1, NB, H), lambda i: (0, i, 0)),
            pl.BlockSpec((1, NB, H), lambda i: (1, i, 0)),
            pl.BlockSpec((1, NB, H), lambda i: (0, i, 0)),
            pl.BlockSpec((1, NB, H), lambda i: (1, i, 0)),
            pl.BlockSpec((D, D), lambda i: (0, 0)),
            pl.BlockSpec((1, D), lambda i: (0, 0)),
            pl.BlockSpec((D, OUT), lambda i: (0, 0)),
            pl.BlockSpec((1, OUT), lambda i: (0, 0)),
            pl.BlockSpec((1, 1, NB), lambda i: (i, 0, 0)),
        ],
        out_specs=pl.BlockSpec((G, OUT), lambda i: (0, 0)),
        out_shape=jax.ShapeDtypeStruct((G, OUT), jnp.float32),
        scratch_shapes=[pltpu.VMEM((G, D), jnp.float32),
                        pltpu.VMEM((G, D), jnp.float32)],
    )(h_both, h_both, a_both, a_both, W2, b2.reshape(1, D), W3,
      b3.reshape(1, OUT), batch.reshape(GRID, 1, NB))


def kernel(x, edge_index, batch, W1, b1, W2, b2, W3, b3):
    idx2d = edge_index.reshape(2, NS, NCHUNK, CHUNK)
    x_both = jnp.stack([x[:, :H], x[:, H:]])

    a1 = _sc_agg(x_both, idx2d)
    h1 = _tc_layer(x_both, a1, W1, b1)
    a2 = _sc_agg(h1, idx2d)
    return _tc_final(h1, a2, W2, b2, W3, b3, batch)
